# baseline (speedup 1.0000x reference)
import math

import jax
import jax.numpy as jnp
from jax import lax
from jax.experimental import pallas as pl

DIM_ = 192
HEADS_ = 4
TOPK_START_ = 0.05
TOPK_END_ = 0.15
ALPHA_MAX_ = 0.35
GAMMA_ = 0.5
WARMUP_ = 1500
STEP_ = 1
EPS_ = 1e-6


def _fused_body(f_ref, post_tok_ref, pos0_ref, q_w_ref, q_b_ref, k_w_ref,
                k_b_ref, v_w_ref, v_b_ref, o_w_ref, o_b_ref, pre_w_ref,
                pre_b_ref, post_w_ref, post_b_ref, g1_w_ref, g2_w_ref,
                g2_b_ref, out_ref, asp_ref, *, kc, alpha):
    C, T = DIM_, f_ref.shape[-1]
    dh = C // HEADS_
    inv_sqrt_dh = 1.0 / math.sqrt(dh)

    f = f_ref[0]

    u = jnp.mean(f, axis=1, keepdims=True)
    msq = jnp.mean(f * f, axis=1, keepdims=True)
    var = msq - u * u
    inv = lax.rsqrt(var + EPS_)
    pre_w = pre_w_ref[0].reshape(C, 1)
    pre_b = pre_b_ref[0].reshape(C, 1)
    x = pre_w * ((f - u) * inv) + pre_b

    xs = x + post_tok_ref[...]
    v_glb = jnp.mean(x, axis=1, keepdims=True)
    xs_cls = v_glb + pos0_ref[0].reshape(C, 1)

    q_vec = jnp.dot(q_w_ref[...], xs_cls, preferred_element_type=jnp.float32)
    q_vec = q_vec + q_b_ref[0].reshape(C, 1)
    hrow = lax.broadcasted_iota(jnp.int32, (HEADS_, C), 0)
    hcol = lax.broadcasted_iota(jnp.int32, (HEADS_, C), 1)
    head_mask = (hcol // dh) == hrow
    q4 = jnp.where(head_mask, q_vec.reshape(1, C), 0.0)

    qk = jnp.dot(q4, k_w_ref[...], preferred_element_type=jnp.float32)
    logits = jnp.dot(qk, xs, preferred_element_type=jnp.float32)
    logits = (logits + jnp.dot(q4, k_b_ref[0].reshape(C, 1),
                               preferred_element_type=jnp.float32)) * inv_sqrt_dh
    logit_cls = (jnp.dot(qk, xs_cls, preferred_element_type=jnp.float32)
                 + jnp.dot(q4, k_b_ref[0].reshape(C, 1),
                           preferred_element_type=jnp.float32)) * inv_sqrt_dh

    m = jnp.maximum(jnp.max(logits, axis=1, keepdims=True), logit_cls)
    e = jnp.exp(logits - m)
    e_cls = jnp.exp(logit_cls - m)
    z = jnp.sum(e, axis=1, keepdims=True) + e_cls
    w_attn = e / z
    w_cls = e_cls / z

    asp = jnp.mean(w_attn, axis=0, keepdims=True)
    asp = asp / (jnp.max(asp) + 1e-6)
    asp_ref[0] = asp

    s = lax.dot_general(xs, w_attn, (((1,), (1,)), ((), ())),
                        preferred_element_type=jnp.float32)
    z_heads = s + xs_cls * w_cls.reshape(1, HEADS_)
    v4 = jnp.dot(v_w_ref[...], z_heads, preferred_element_type=jnp.float32)
    sel = (lax.broadcasted_iota(jnp.int32, (C, HEADS_), 0) // dh
           == lax.broadcasted_iota(jnp.int32, (C, HEADS_), 1))
    pooled = jnp.sum(jnp.where(sel, v4, 0.0), axis=1, keepdims=True)
    pooled = pooled + v_b_ref[0].reshape(C, 1)
    pooled = jnp.dot(o_w_ref[...], pooled, preferred_element_type=jnp.float32)
    pooled = pooled + o_b_ref[0].reshape(C, 1)

    sal = jnp.mean(x * x, axis=0, keepdims=True)
    bits = lax.bitcast_convert_type(sal, jnp.int32)

    def bs_body(_, carry):
        lo, hi = carry
        mid = lo + ((hi - lo + 1) >> 1)
        cnt = jnp.sum((bits >= mid).astype(jnp.int32), keepdims=True,
                      axis=(0, 1))
        big = cnt >= kc
        return jnp.where(big, mid, lo), jnp.where(big, hi, mid - 1)

    lo0 = jnp.zeros((1, 1), jnp.int32)
    hi0 = jnp.max(bits, keepdims=True, axis=(0, 1))
    tau, _ = lax.fori_loop(0, 31, bs_body, (lo0, hi0))
    gt = bits > tau
    eqm = bits == tau
    c_gt = jnp.sum(gt.astype(jnp.int32), keepdims=True, axis=(0, 1))
    n_eq = jnp.maximum(jnp.sum(eqm.astype(jnp.int32), keepdims=True,
                               axis=(0, 1)), 1)
    w_eq = (kc - c_gt).astype(jnp.float32) / n_eq.astype(jnp.float32)
    wsel = jnp.where(gt, 1.0, jnp.where(eqm, w_eq, 0.0))
    refine = lax.dot_general(x, wsel, (((1,), (1,)), ((), ())),
                             preferred_element_type=jnp.float32) / kc

    v_fg = 0.8 * pooled + 0.2 * refine
    v_fused = GAMMA_ * v_glb + (1.0 - GAMMA_) * v_fg
    h1 = jnp.dot(g1_w_ref[...], v_fused, preferred_element_type=jnp.float32)
    h1 = jnp.maximum(h1, 0.0)
    g = jnp.dot(g2_w_ref[...], h1, preferred_element_type=jnp.float32)
    g = g + g2_b_ref[0].reshape(C, 1)
    gate = 1.0 / (1.0 + jnp.exp(-g))

    fused = x * gate * (1.0 + asp)
    u2 = jnp.mean(fused, axis=1, keepdims=True)
    msq2 = jnp.mean(fused * fused, axis=1, keepdims=True)
    var2 = msq2 - u2 * u2
    inv2 = lax.rsqrt(var2 + EPS_)
    post_w = post_w_ref[0].reshape(C, 1)
    post_b = post_b_ref[0].reshape(C, 1)
    ln2 = post_w * ((fused - u2) * inv2) + post_b
    out_ref[0] = f + alpha * (ln2 - f)


def kernel(feat_2d, pos, q_w, q_b, k_w, k_b, v_w, v_b, o_w, o_b,
           pre_w, pre_b, post_w, post_b, g1_w, g2_w, g2_b):
    B, C, H, W = feat_2d.shape
    T = H * W

    t = float(min(STEP_, WARMUP_))
    ratio = 0.5 * (1.0 - math.cos(math.pi * t / WARMUP_))
    alpha = ratio * ALPHA_MAX_
    topk_ratio = TOPK_START_ + (TOPK_END_ - TOPK_START_) * ratio
    kc = max(1, int(T * topk_ratio))

    fr = feat_2d.reshape(B, C, T)
    posT = pos.T
    pos0 = posT[:, :1].reshape(1, C)
    post_tok = posT[:, 1:]

    row = lambda v: v.reshape(1, C)
    full = lambda shape: pl.BlockSpec(shape, lambda b: (0,) * len(shape))

    import functools
    body = functools.partial(_fused_body, kc=kc, alpha=alpha)

    out, asp = pl.pallas_call(
        body,
        grid=(B,),
        in_specs=[
            pl.BlockSpec((1, C, T), lambda b: (b, 0, 0)),
            full((C, T)),
            full((1, C)),
            full((C, C)),
            full((1, C)),
            full((C, C)),
            full((1, C)),
            full((C, C)),
            full((1, C)),
            full((C, C)),
            full((1, C)),
            full((1, C)),
            full((1, C)),
            full((1, C)),
            full((1, C)),
            full((C // 4, C)),
            full((C, C // 4)),
            full((1, C)),
        ],
        out_specs=[
            pl.BlockSpec((1, C, T), lambda b: (b, 0, 0)),
            pl.BlockSpec((1, 1, T), lambda b: (b, 0, 0)),
        ],
        out_shape=[
            jax.ShapeDtypeStruct((B, C, T), jnp.float32),
            jax.ShapeDtypeStruct((B, 1, T), jnp.float32),
        ],
    )(fr, post_tok, pos0, q_w, row(q_b), k_w, row(k_b), v_w, row(v_b),
      o_w, row(o_b), row(pre_w), row(pre_b), row(post_w), row(post_b),
      g1_w, g2_w, row(g2_b))

    return out.reshape(B, C, H, W), asp.reshape(B, 1, H, W)


# speedup vs baseline: 1.6438x; 1.6438x over previous
import functools
import math

import jax
import jax.numpy as jnp
from jax import lax
from jax.experimental import pallas as pl

DIM_ = 192
HEADS_ = 4
TOPK_START_ = 0.05
TOPK_END_ = 0.15
ALPHA_MAX_ = 0.35
GAMMA_ = 0.5
WARMUP_ = 1500
STEP_ = 1
EPS_ = 1e-6
BPP_ = 2


def _fused_body(f_ref, post_tok_ref, qk_s_ref, lconst_ref, lcls_ref,
                xs_cls_ref, v_w_ref, v_b_ref, o_w_ref, o_b_ref, pre_w_ref,
                pre_b_ref, post_w_ref, post_b_ref, g1_w_ref, g2_w_ref,
                g2_b_ref, out_ref, asp_ref, *, kc, alpha):
    C, T = DIM_, f_ref.shape[-1]
    dh = C // HEADS_

    pre_w = pre_w_ref[0].reshape(C, 1)
    pre_b = pre_b_ref[0].reshape(C, 1)
    post_w = post_w_ref[0].reshape(C, 1)
    post_b = post_b_ref[0].reshape(C, 1)
    xs_cls = xs_cls_ref[0].reshape(C, 1)
    lcls = lcls_ref[...].reshape(HEADS_, 1)
    sel = (lax.broadcasted_iota(jnp.int32, (C, HEADS_), 0) // dh
           == lax.broadcasted_iota(jnp.int32, (C, HEADS_), 1))

    xs_list, fs, bits_list, asps, pooleds, vglbs = [], [], [], [], [], []
    for i in range(BPP_):
        f = f_ref[i]

        u = jnp.mean(f, axis=1, keepdims=True)
        msq = jnp.mean(f * f, axis=1, keepdims=True)
        inv = lax.rsqrt(msq - u * u + EPS_)
        a1 = pre_w * inv
        x = f * a1 + (pre_b - u * a1)

        logits = jnp.dot(qk_s_ref[...], x,
                         preferred_element_type=jnp.float32) + lconst_ref[...]
        m = jnp.maximum(jnp.max(logits, axis=1, keepdims=True), lcls)
        e = jnp.exp(logits - m)
        e_cls = jnp.exp(lcls - m)
        z = jnp.sum(e, axis=1, keepdims=True) + e_cls
        w_attn = e / z
        w_cls = e_cls / z

        asp = jnp.sum(w_attn, axis=0, keepdims=True) * (1.0 / HEADS_)
        asp = asp / (jnp.max(asp) + 1e-6)
        asps.append(asp)
        asp_ref[i, 0] = asp[0]

        s = (lax.dot_general(x, w_attn, (((1,), (1,)), ((), ())),
                             preferred_element_type=jnp.float32)
             + lax.dot_general(post_tok_ref[...], w_attn,
                               (((1,), (1,)), ((), ())),
                               preferred_element_type=jnp.float32))
        z_heads = s + xs_cls * w_cls.reshape(1, HEADS_)
        v4 = jnp.dot(v_w_ref[...], z_heads, preferred_element_type=jnp.float32)
        pooled = jnp.sum(jnp.where(sel, v4, 0.0), axis=1, keepdims=True)
        pooled = pooled + v_b_ref[0].reshape(C, 1)
        pooled = jnp.dot(o_w_ref[...], pooled,
                         preferred_element_type=jnp.float32)
        pooled = pooled + o_b_ref[0].reshape(C, 1)
        pooleds.append(pooled)

        vglbs.append(jnp.mean(x, axis=1, keepdims=True))

        sal = jnp.mean(x * x, axis=0, keepdims=True)
        bits_list.append(lax.bitcast_convert_type(sal, jnp.int32))
        fs.append(f)
        xs_list.append(x)

    bits = jnp.concatenate(bits_list, axis=0)

    def bs_body(_, carry):
        lo, hi = carry
        mid = lo + ((hi - lo + 1) >> 1)
        cnt = jnp.sum((bits >= mid).astype(jnp.int32), axis=1, keepdims=True)
        big = cnt >= kc
        return jnp.where(big, mid, lo), jnp.where(big, hi, mid - 1)

    lo0 = jnp.zeros((BPP_, 1), jnp.int32)
    hi0 = jnp.max(bits, axis=1, keepdims=True)
    tau, _ = lax.fori_loop(0, 31, bs_body, (lo0, hi0))
    gt = bits > tau
    eqm = bits == tau
    c_gt = jnp.sum(gt.astype(jnp.int32), axis=1, keepdims=True)
    n_eq = jnp.maximum(jnp.sum(eqm.astype(jnp.int32), axis=1, keepdims=True), 1)
    w_eq = (kc - c_gt).astype(jnp.float32) / n_eq.astype(jnp.float32)
    wsel = jnp.where(gt, 1.0, jnp.where(eqm, w_eq, 0.0))

    for i in range(BPP_):
        f, x = fs[i], xs_list[i]
        refine = lax.dot_general(x, wsel[i:i + 1],
                                 (((1,), (1,)), ((), ())),
                                 preferred_element_type=jnp.float32)
        refine = refine * (1.0 / kc)

        v_fg = 0.8 * pooleds[i] + 0.2 * refine
        v_fused = GAMMA_ * vglbs[i] + (1.0 - GAMMA_) * v_fg
        h1 = jnp.dot(g1_w_ref[...], v_fused,
                     preferred_element_type=jnp.float32)
        h1 = jnp.maximum(h1, 0.0)
        g = jnp.dot(g2_w_ref[...], h1, preferred_element_type=jnp.float32)
        g = g + g2_b_ref[0].reshape(C, 1)
        gate = 1.0 / (1.0 + jnp.exp(-g))

        fused = (x * (1.0 + asps[i])) * gate
        u2 = jnp.mean(fused, axis=1, keepdims=True)
        msq2 = jnp.mean(fused * fused, axis=1, keepdims=True)
        inv2 = lax.rsqrt(msq2 - u2 * u2 + EPS_)
        a2 = alpha * (post_w * inv2)
        b2 = alpha * post_b - u2 * a2
        out_ref[i] = f * (1.0 - alpha) + (fused * a2 + b2)


def kernel(feat_2d, pos, q_w, q_b, k_w, k_b, v_w, v_b, o_w, o_b,
           pre_w, pre_b, post_w, post_b, g1_w, g2_w, g2_b):
    B, C, H, W = feat_2d.shape
    T = H * W
    dh = C // HEADS_

    t = float(min(STEP_, WARMUP_))
    ratio = 0.5 * (1.0 - math.cos(math.pi * t / WARMUP_))
    alpha = ratio * ALPHA_MAX_
    topk_ratio = TOPK_START_ + (TOPK_END_ - TOPK_START_) * ratio
    kc = max(1, int(T * topk_ratio))

    fr = feat_2d.reshape(B, C, T)
    posT = pos.T
    pos0 = posT[:, :1]
    post_tok = posT[:, 1:]

    xs_cls = pre_b.reshape(C, 1) + pos0
    q_vec = q_w @ xs_cls + q_b.reshape(C, 1)
    head_mask = (jnp.arange(C)[None, :] // dh) == jnp.arange(HEADS_)[:, None]
    q4 = jnp.where(head_mask, q_vec.reshape(1, C), 0.0)
    inv_sqrt_dh = 1.0 / math.sqrt(dh)
    qk_s = (q4 @ k_w) * inv_sqrt_dh
    kb_term = (q4 @ k_b.reshape(C, 1)) * inv_sqrt_dh
    lconst = qk_s @ post_tok + kb_term
    lcls = (qk_s @ xs_cls + kb_term).reshape(1, HEADS_)

    row = lambda v: v.reshape(1, C)
    full = lambda shape: pl.BlockSpec(shape, lambda b: (0,) * len(shape))

    body = functools.partial(_fused_body, kc=kc, alpha=alpha)

    out, asp = pl.pallas_call(
        body,
        grid=(B // BPP_,),
        in_specs=[
            pl.BlockSpec((BPP_, C, T), lambda b: (b, 0, 0)),
            full((C, T)),
            full((HEADS_, C)),
            full((HEADS_, T)),
            full((1, HEADS_)),
            full((1, C)),
            full((C, C)),
            full((1, C)),
            full((C, C)),
            full((1, C)),
            full((1, C)),
            full((1, C)),
            full((1, C)),
            full((1, C)),
            full((C // 4, C)),
            full((C, C // 4)),
            full((1, C)),
        ],
        out_specs=[
            pl.BlockSpec((BPP_, C, T), lambda b: (b, 0, 0)),
            pl.BlockSpec((BPP_, 1, T), lambda b: (b, 0, 0)),
        ],
        out_shape=[
            jax.ShapeDtypeStruct((B, C, T), jnp.float32),
            jax.ShapeDtypeStruct((B, 1, T), jnp.float32),
        ],
    )(fr, post_tok, qk_s, lconst, lcls, xs_cls.reshape(1, C), v_w, row(v_b),
      o_w, row(o_b), row(pre_w), row(pre_b), row(post_w), row(post_b),
      g1_w, g2_w, row(g2_b))

    return out.reshape(B, C, H, W), asp.reshape(B, 1, H, W)


# speedup vs baseline: 1.9662x; 1.1961x over previous
import functools
import math

import jax
import jax.numpy as jnp
from jax import lax
from jax.experimental import pallas as pl

DIM_ = 192
HEADS_ = 4
TOPK_START_ = 0.05
TOPK_END_ = 0.15
ALPHA_MAX_ = 0.35
GAMMA_ = 0.5
WARMUP_ = 1500
STEP_ = 1
EPS_ = 1e-6
BPP_ = 4


def _fused_body(f_ref, post_tok_ref, qk_s_ref, lconst_ref, lcls_ref,
                xs_cls_ref, v_w_ref, v_b_ref, o_w_ref, o_b_ref, pre_w_ref,
                pre_b_ref, post_w_ref, post_b_ref, g1_w_ref, g2_w_ref,
                g2_b_ref, out_ref, asp_ref, *, kc, alpha):
    C, T = DIM_, f_ref.shape[-1]
    dh = C // HEADS_

    pre_w = pre_w_ref[0].reshape(C, 1)
    pre_b = pre_b_ref[0].reshape(C, 1)
    post_w = post_w_ref[0].reshape(C, 1)
    post_b = post_b_ref[0].reshape(C, 1)
    xs_cls = xs_cls_ref[0].reshape(C, 1)
    lcls = lcls_ref[...].reshape(HEADS_, 1)
    sel = (lax.broadcasted_iota(jnp.int32, (C, HEADS_), 0) // dh
           == lax.broadcasted_iota(jnp.int32, (C, HEADS_), 1))

    xs_list, fs, bits_list, asps, pooleds, vglbs = [], [], [], [], [], []
    for i in range(BPP_):
        f = f_ref[i]

        u = jnp.mean(f, axis=1, keepdims=True)
        msq = jnp.mean(f * f, axis=1, keepdims=True)
        inv = lax.rsqrt(msq - u * u + EPS_)
        a1 = pre_w * inv
        x = f * a1 + (pre_b - u * a1)

        logits = jnp.dot(qk_s_ref[...], x,
                         preferred_element_type=jnp.float32) + lconst_ref[...]
        m = jnp.maximum(jnp.max(logits, axis=1, keepdims=True), lcls)
        e = jnp.exp(logits - m)
        e_cls = jnp.exp(lcls - m)
        z = jnp.sum(e, axis=1, keepdims=True) + e_cls
        w_attn = e / z
        w_cls = e_cls / z

        asp = jnp.sum(w_attn, axis=0, keepdims=True) * (1.0 / HEADS_)
        asp = asp / (jnp.max(asp) + 1e-6)
        asps.append(asp)
        asp_ref[i, 0] = asp[0]

        s = (lax.dot_general(x, w_attn, (((1,), (1,)), ((), ())),
                             preferred_element_type=jnp.float32)
             + lax.dot_general(post_tok_ref[...], w_attn,
                               (((1,), (1,)), ((), ())),
                               preferred_element_type=jnp.float32))
        z_heads = s + xs_cls * w_cls.reshape(1, HEADS_)
        v4 = jnp.dot(v_w_ref[...], z_heads, preferred_element_type=jnp.float32)
        pooled = jnp.sum(jnp.where(sel, v4, 0.0), axis=1, keepdims=True)
        pooled = pooled + v_b_ref[0].reshape(C, 1)
        pooled = jnp.dot(o_w_ref[...], pooled,
                         preferred_element_type=jnp.float32)
        pooled = pooled + o_b_ref[0].reshape(C, 1)
        pooleds.append(pooled)

        vglbs.append(jnp.mean(x, axis=1, keepdims=True))

        sal = jnp.mean(x * x, axis=0, keepdims=True)
        bits_list.append(lax.bitcast_convert_type(sal, jnp.int32))
        fs.append(f)
        xs_list.append(x)

    bits = jnp.concatenate(bits_list, axis=0)

    def bs_body(_, carry):
        lo, hi = carry
        mid = lo + ((hi - lo + 1) >> 1)
        cnt = jnp.sum((bits >= mid).astype(jnp.int32), axis=1, keepdims=True)
        big = cnt >= kc
        return jnp.where(big, mid, lo), jnp.where(big, hi, mid - 1)

    lo0 = jnp.zeros((BPP_, 1), jnp.int32)
    hi0 = jnp.max(bits, axis=1, keepdims=True)
    tau, _ = lax.fori_loop(0, 31, bs_body, (lo0, hi0))
    gt = bits > tau
    eqm = bits == tau
    c_gt = jnp.sum(gt.astype(jnp.int32), axis=1, keepdims=True)
    n_eq = jnp.maximum(jnp.sum(eqm.astype(jnp.int32), axis=1, keepdims=True), 1)
    w_eq = (kc - c_gt).astype(jnp.float32) / n_eq.astype(jnp.float32)
    wsel = jnp.where(gt, 1.0, jnp.where(eqm, w_eq, 0.0))

    for i in range(BPP_):
        f, x = fs[i], xs_list[i]
        refine = lax.dot_general(x, wsel[i:i + 1],
                                 (((1,), (1,)), ((), ())),
                                 preferred_element_type=jnp.float32)
        refine = refine * (1.0 / kc)

        v_fg = 0.8 * pooleds[i] + 0.2 * refine
        v_fused = GAMMA_ * vglbs[i] + (1.0 - GAMMA_) * v_fg
        h1 = jnp.dot(g1_w_ref[...], v_fused,
                     preferred_element_type=jnp.float32)
        h1 = jnp.maximum(h1, 0.0)
        g = jnp.dot(g2_w_ref[...], h1, preferred_element_type=jnp.float32)
        g = g + g2_b_ref[0].reshape(C, 1)
        gate = 1.0 / (1.0 + jnp.exp(-g))

        fused = (x * (1.0 + asps[i])) * gate
        u2 = jnp.mean(fused, axis=1, keepdims=True)
        msq2 = jnp.mean(fused * fused, axis=1, keepdims=True)
        inv2 = lax.rsqrt(msq2 - u2 * u2 + EPS_)
        a2 = alpha * (post_w * inv2)
        b2 = alpha * post_b - u2 * a2
        out_ref[i] = f * (1.0 - alpha) + (fused * a2 + b2)


def kernel(feat_2d, pos, q_w, q_b, k_w, k_b, v_w, v_b, o_w, o_b,
           pre_w, pre_b, post_w, post_b, g1_w, g2_w, g2_b):
    B, C, H, W = feat_2d.shape
    T = H * W
    dh = C // HEADS_

    t = float(min(STEP_, WARMUP_))
    ratio = 0.5 * (1.0 - math.cos(math.pi * t / WARMUP_))
    alpha = ratio * ALPHA_MAX_
    topk_ratio = TOPK_START_ + (TOPK_END_ - TOPK_START_) * ratio
    kc = max(1, int(T * topk_ratio))

    fr = feat_2d.reshape(B, C, T)
    posT = pos.T
    pos0 = posT[:, :1]
    post_tok = posT[:, 1:]

    xs_cls = pre_b.reshape(C, 1) + pos0
    q_vec = q_w @ xs_cls + q_b.reshape(C, 1)
    head_mask = (jnp.arange(C)[None, :] // dh) == jnp.arange(HEADS_)[:, None]
    q4 = jnp.where(head_mask, q_vec.reshape(1, C), 0.0)
    inv_sqrt_dh = 1.0 / math.sqrt(dh)
    qk_s = (q4 @ k_w) * inv_sqrt_dh
    kb_term = (q4 @ k_b.reshape(C, 1)) * inv_sqrt_dh
    lconst = qk_s @ post_tok + kb_term
    lcls = (qk_s @ xs_cls + kb_term).reshape(1, HEADS_)

    row = lambda v: v.reshape(1, C)
    full = lambda shape: pl.BlockSpec(shape, lambda b: (0,) * len(shape))

    body = functools.partial(_fused_body, kc=kc, alpha=alpha)

    out, asp = pl.pallas_call(
        body,
        grid=(B // BPP_,),
        in_specs=[
            pl.BlockSpec((BPP_, C, T), lambda b: (b, 0, 0)),
            full((C, T)),
            full((HEADS_, C)),
            full((HEADS_, T)),
            full((1, HEADS_)),
            full((1, C)),
            full((C, C)),
            full((1, C)),
            full((C, C)),
            full((1, C)),
            full((1, C)),
            full((1, C)),
            full((1, C)),
            full((1, C)),
            full((C // 4, C)),
            full((C, C // 4)),
            full((1, C)),
        ],
        out_specs=[
            pl.BlockSpec((BPP_, C, T), lambda b: (b, 0, 0)),
            pl.BlockSpec((BPP_, 1, T), lambda b: (b, 0, 0)),
        ],
        out_shape=[
            jax.ShapeDtypeStruct((B, C, T), jnp.float32),
            jax.ShapeDtypeStruct((B, 1, T), jnp.float32),
        ],
    )(fr, post_tok, qk_s, lconst, lcls, xs_cls.reshape(1, C), v_w, row(v_b),
      o_w, row(o_b), row(pre_w), row(pre_b), row(post_w), row(post_b),
      g1_w, g2_w, row(g2_b))

    return out.reshape(B, C, H, W), asp.reshape(B, 1, H, W)


# speedup vs baseline: 2.1262x; 1.0814x over previous
import functools
import math

import jax
import jax.numpy as jnp
from jax import lax
from jax.experimental import pallas as pl

DIM_ = 192
HEADS_ = 4
TOPK_START_ = 0.05
TOPK_END_ = 0.15
ALPHA_MAX_ = 0.35
GAMMA_ = 0.5
WARMUP_ = 1500
STEP_ = 1
EPS_ = 1e-6
BPP_ = 8


def _fused_body(f_ref, post_tok_ref, qk_s_ref, lconst_ref, lcls_ref,
                xs_cls_ref, v_w_ref, v_b_ref, o_w_ref, o_b_ref, pre_w_ref,
                pre_b_ref, post_w_ref, post_b_ref, g1_w_ref, g2_w_ref,
                g2_b_ref, out_ref, asp_ref, *, kc, alpha):
    C, T = DIM_, f_ref.shape[-1]
    dh = C // HEADS_

    pre_w = pre_w_ref[0].reshape(C, 1)
    pre_b = pre_b_ref[0].reshape(C, 1)
    post_w = post_w_ref[0].reshape(C, 1)
    post_b = post_b_ref[0].reshape(C, 1)
    xs_cls = xs_cls_ref[0].reshape(C, 1)
    lcls = lcls_ref[...].reshape(HEADS_, 1)
    sel = (lax.broadcasted_iota(jnp.int32, (C, HEADS_), 0) // dh
           == lax.broadcasted_iota(jnp.int32, (C, HEADS_), 1))

    xs_list, fs, bits_list, asps, pooleds, vglbs = [], [], [], [], [], []
    for i in range(BPP_):
        f = f_ref[i]

        u = jnp.mean(f, axis=1, keepdims=True)
        msq = jnp.mean(f * f, axis=1, keepdims=True)
        inv = lax.rsqrt(msq - u * u + EPS_)
        a1 = pre_w * inv
        x = f * a1 + (pre_b - u * a1)

        logits = jnp.dot(qk_s_ref[...], x,
                         preferred_element_type=jnp.float32) + lconst_ref[...]
        m = jnp.maximum(jnp.max(logits, axis=1, keepdims=True), lcls)
        e = jnp.exp(logits - m)
        e_cls = jnp.exp(lcls - m)
        z = jnp.sum(e, axis=1, keepdims=True) + e_cls
        w_attn = e / z
        w_cls = e_cls / z

        asp = jnp.sum(w_attn, axis=0, keepdims=True) * (1.0 / HEADS_)
        asp = asp / (jnp.max(asp) + 1e-6)
        asps.append(asp)
        asp_ref[i, 0] = asp[0]

        s = (lax.dot_general(x, w_attn, (((1,), (1,)), ((), ())),
                             preferred_element_type=jnp.float32)
             + lax.dot_general(post_tok_ref[...], w_attn,
                               (((1,), (1,)), ((), ())),
                               preferred_element_type=jnp.float32))
        z_heads = s + xs_cls * w_cls.reshape(1, HEADS_)
        v4 = jnp.dot(v_w_ref[...], z_heads, preferred_element_type=jnp.float32)
        pooled = jnp.sum(jnp.where(sel, v4, 0.0), axis=1, keepdims=True)
        pooled = pooled + v_b_ref[0].reshape(C, 1)
        pooled = jnp.dot(o_w_ref[...], pooled,
                         preferred_element_type=jnp.float32)
        pooled = pooled + o_b_ref[0].reshape(C, 1)
        pooleds.append(pooled)

        vglbs.append(jnp.mean(x, axis=1, keepdims=True))

        sal = jnp.mean(x * x, axis=0, keepdims=True)
        bits_list.append(lax.bitcast_convert_type(sal, jnp.int32))
        fs.append(f)
        xs_list.append(x)

    bits = jnp.concatenate(bits_list, axis=0)

    def bs_body(_, carry):
        lo, hi = carry
        mid = lo + ((hi - lo + 1) >> 1)
        cnt = jnp.sum((bits >= mid).astype(jnp.int32), axis=1, keepdims=True)
        big = cnt >= kc
        return jnp.where(big, mid, lo), jnp.where(big, hi, mid - 1)

    lo0 = jnp.zeros((BPP_, 1), jnp.int32)
    hi0 = jnp.max(bits, axis=1, keepdims=True)
    tau, _ = lax.fori_loop(0, 31, bs_body, (lo0, hi0))
    gt = bits > tau
    eqm = bits == tau
    c_gt = jnp.sum(gt.astype(jnp.int32), axis=1, keepdims=True)
    n_eq = jnp.maximum(jnp.sum(eqm.astype(jnp.int32), axis=1, keepdims=True), 1)
    w_eq = (kc - c_gt).astype(jnp.float32) / n_eq.astype(jnp.float32)
    wsel = jnp.where(gt, 1.0, jnp.where(eqm, w_eq, 0.0))

    for i in range(BPP_):
        f, x = fs[i], xs_list[i]
        refine = lax.dot_general(x, wsel[i:i + 1],
                                 (((1,), (1,)), ((), ())),
                                 preferred_element_type=jnp.float32)
        refine = refine * (1.0 / kc)

        v_fg = 0.8 * pooleds[i] + 0.2 * refine
        v_fused = GAMMA_ * vglbs[i] + (1.0 - GAMMA_) * v_fg
        h1 = jnp.dot(g1_w_ref[...], v_fused,
                     preferred_element_type=jnp.float32)
        h1 = jnp.maximum(h1, 0.0)
        g = jnp.dot(g2_w_ref[...], h1, preferred_element_type=jnp.float32)
        g = g + g2_b_ref[0].reshape(C, 1)
        gate = 1.0 / (1.0 + jnp.exp(-g))

        fused = (x * (1.0 + asps[i])) * gate
        u2 = jnp.mean(fused, axis=1, keepdims=True)
        msq2 = jnp.mean(fused * fused, axis=1, keepdims=True)
        inv2 = lax.rsqrt(msq2 - u2 * u2 + EPS_)
        a2 = alpha * (post_w * inv2)
        b2 = alpha * post_b - u2 * a2
        out_ref[i] = f * (1.0 - alpha) + (fused * a2 + b2)


def kernel(feat_2d, pos, q_w, q_b, k_w, k_b, v_w, v_b, o_w, o_b,
           pre_w, pre_b, post_w, post_b, g1_w, g2_w, g2_b):
    B, C, H, W = feat_2d.shape
    T = H * W
    dh = C // HEADS_

    t = float(min(STEP_, WARMUP_))
    ratio = 0.5 * (1.0 - math.cos(math.pi * t / WARMUP_))
    alpha = ratio * ALPHA_MAX_
    topk_ratio = TOPK_START_ + (TOPK_END_ - TOPK_START_) * ratio
    kc = max(1, int(T * topk_ratio))

    fr = feat_2d.reshape(B, C, T)
    posT = pos.T
    pos0 = posT[:, :1]
    post_tok = posT[:, 1:]

    xs_cls = pre_b.reshape(C, 1) + pos0
    q_vec = q_w @ xs_cls + q_b.reshape(C, 1)
    head_mask = (jnp.arange(C)[None, :] // dh) == jnp.arange(HEADS_)[:, None]
    q4 = jnp.where(head_mask, q_vec.reshape(1, C), 0.0)
    inv_sqrt_dh = 1.0 / math.sqrt(dh)
    qk_s = (q4 @ k_w) * inv_sqrt_dh
    kb_term = (q4 @ k_b.reshape(C, 1)) * inv_sqrt_dh
    lconst = qk_s @ post_tok + kb_term
    lcls = (qk_s @ xs_cls + kb_term).reshape(1, HEADS_)

    row = lambda v: v.reshape(1, C)
    full = lambda shape: pl.BlockSpec(shape, lambda b: (0,) * len(shape))

    body = functools.partial(_fused_body, kc=kc, alpha=alpha)

    out, asp = pl.pallas_call(
        body,
        grid=(B // BPP_,),
        in_specs=[
            pl.BlockSpec((BPP_, C, T), lambda b: (b, 0, 0)),
            full((C, T)),
            full((HEADS_, C)),
            full((HEADS_, T)),
            full((1, HEADS_)),
            full((1, C)),
            full((C, C)),
            full((1, C)),
            full((C, C)),
            full((1, C)),
            full((1, C)),
            full((1, C)),
            full((1, C)),
            full((1, C)),
            full((C // 4, C)),
            full((C, C // 4)),
            full((1, C)),
        ],
        out_specs=[
            pl.BlockSpec((BPP_, C, T), lambda b: (b, 0, 0)),
            pl.BlockSpec((BPP_, 1, T), lambda b: (b, 0, 0)),
        ],
        out_shape=[
            jax.ShapeDtypeStruct((B, C, T), jnp.float32),
            jax.ShapeDtypeStruct((B, 1, T), jnp.float32),
        ],
    )(fr, post_tok, qk_s, lconst, lcls, xs_cls.reshape(1, C), v_w, row(v_b),
      o_w, row(o_b), row(pre_w), row(pre_b), row(post_w), row(post_b),
      g1_w, g2_w, row(g2_b))

    return out.reshape(B, C, H, W), asp.reshape(B, 1, H, W)
